# baseline (speedup 1.0000x reference)
import jax
import jax.numpy as jnp
from jax import lax
from jax.experimental import pallas as pl
from jax.experimental.pallas import tpu as pltpu
from jax.experimental.pallas import tpu_sc as plsc

POOL_H = 7
POOL_W = 7
NPX = POOL_H * POOL_W
NIDX = 2 * NPX
C = 256
NW = 32
NBOX = 1000
BPW = 32


def _roi_body(boxes_hbm, meta_hbm, ctab_hbm, f2, f3, f4, f5, out_hbm,
              bx_v, meta_v, ct_v, y0t, y1t, x0t, x1t,
              wy0, wx0, wy1, wx1,
              idx_a0, idx_b0, idx_a1, idx_b1,
              rows_a0, rows_b0, rows_a1, rows_b1,
              out_v0, out_v1, sem_g0, sem_g1, sem_o0, sem_o1):
    cid = lax.axis_index("c")
    sid = lax.axis_index("s")
    wid = sid * 2 + cid
    base = wid * 31 + jnp.minimum(wid, 8)
    cnt = 31 + (wid < 8).astype(jnp.int32)

    pltpu.sync_copy(boxes_hbm.at[pl.ds(base * 8, BPW * 8 + 16)], bx_v)
    pltpu.sync_copy(meta_hbm, meta_v)
    pltpu.sync_copy(ctab_hbm, ct_v)
    mv = meta_v[pl.ds(0, 16)]
    area = mv[4] * mv[5]
    th3 = jnp.float32(224.0 * 224.0 * 0.125)
    th4 = jnp.float32(224.0 * 224.0 * 0.5)
    th5 = jnp.float32(224.0 * 224.0 * 2.0)
    lanes = lax.broadcasted_iota(jnp.int32, (16,), 0)
    lanesf = lanes.astype(jnp.float32)

    def prefetch(i, idx_a, idx_b, rows_a, rows_b, wysp, wxsp, sem_g):
        bv = bx_v[pl.ds(i * 8, 16)]
        y1 = bv[0]
        x1 = bv[1]
        y2 = bv[2]
        x2 = bv[3]
        bh = y2 - y1
        bw = x2 - x1
        hw = bh * bw * area
        lvl = (2 + (hw > th3).astype(jnp.int32)
               + (hw > th4).astype(jnp.int32)
               + (hw > th5).astype(jnp.int32))
        wdim = lax.shift_right_logical(jnp.int32(256), lvl - 2)
        wm1 = wdim - 1
        wm1f = wm1.astype(jnp.float32)

        ysv = y1 * wm1f + lanesf * (bh * wm1f * (1.0 / 6.0))
        xsv = x1 * wm1f + lanesf * (bw * wm1f * (1.0 / 6.0))
        y0i = ysv.astype(jnp.int32)
        x0i = xsv.astype(jnp.int32)
        wyv = ysv - y0i.astype(jnp.float32)
        wxv = xsv - x0i.astype(jnp.float32)
        y0c = jnp.maximum(jnp.minimum(y0i, wm1), 0)
        x0c = jnp.maximum(jnp.minimum(x0i, wm1), 0)
        y0t[...] = y0c
        y1t[...] = jnp.minimum(y0c + 1, wm1)
        x0t[...] = x0c
        x1t[...] = jnp.minimum(x0c + 1, wm1)
        for k in range(POOL_H):
            wysp[k, :] = jnp.full((16,), wyv[k])
            wxsp[k, :] = jnp.full((16,), wxv[k])

        for ytab, idxref in ((y0t, idx_a), (y1t, idx_b)):
            for j in range(7):
                iy = ct_v[pl.ds(j * 64, 16)]
                ix = ct_v[pl.ds(j * 64 + 16, 16)]
                is_tl = ct_v[pl.ds(j * 64 + 32, 16)] > 0
                yv = plsc.load_gather(ytab, [iy])
                xv = jnp.where(is_tl,
                               plsc.load_gather(x0t, [ix]),
                               plsc.load_gather(x1t, [ix]))
                idxv = yv * wdim + xv
                if j < 6:
                    idxref[pl.ds(j * 16, 16)] = idxv
                else:
                    gcv = ct_v[pl.ds(j * 64 + 48, 16)]
                    plsc.store_scatter(idxref, [gcv], idxv, mask=lanes < 2)

        for level, fmap in ((2, f2), (3, f3), (4, f4), (5, f5)):
            @pl.when(lvl == level)
            def _():
                pltpu.async_copy(fmap.at[idx_a], rows_a, sem_g)
                pltpu.async_copy(fmap.at[idx_b], rows_b, sem_g)

    def consume(i, rows_a, rows_b, wysp, wxsp, out_v, sem_g, sem_o):
        pltpu.make_async_copy(f2.at[idx_a0], rows_a, sem_g).wait()
        pltpu.make_async_copy(f2.at[idx_b0], rows_b, sem_g).wait()

        @pl.when(i >= 2)
        def _():
            pltpu.make_async_copy(out_v, out_hbm.at[pl.ds(0, NPX)],
                                  sem_o).wait()

        def iy_body(iy, c1):
            wyv = wysp[iy, :]
            omy = 1.0 - wyv

            def ix_body(ix, c2):
                wxv = wxsp[ix, :]
                omx = 1.0 - wxv
                w00 = omy * omx
                w01 = omy * wxv
                w10 = wyv * omx
                w11 = wyv * wxv
                p = iy * POOL_W + ix
                q = p + NPX
                for ck in range(C // 16):
                    sl = pl.ds(ck * 16, 16)
                    out_v[p, sl] = (rows_a[p, sl] * w00 + rows_a[q, sl] * w01
                                    + rows_b[p, sl] * w10 + rows_b[q, sl] * w11)
                return c2
            lax.fori_loop(0, POOL_W, ix_body, 0)
            return c1
        lax.fori_loop(0, POOL_H, iy_body, 0)

        pltpu.async_copy(out_v, out_hbm.at[pl.ds((base + i) * NPX, NPX)],
                         sem_o)

    prefetch(0, idx_a0, idx_b0, rows_a0, rows_b0, wy0, wx0, sem_g0)

    def box_body(i, carry):
        par0 = (i & 1) == 0
        nxt = i + 1

        @pl.when(nxt < cnt)
        def _():
            @pl.when(par0)
            def _():
                prefetch(nxt, idx_a1, idx_b1, rows_a1, rows_b1,
                         wy1, wx1, sem_g1)

            @pl.when(jnp.logical_not(par0))
            def _():
                prefetch(nxt, idx_a0, idx_b0, rows_a0, rows_b0,
                         wy0, wx0, sem_g0)

        @pl.when(par0)
        def _():
            consume(i, rows_a0, rows_b0, wy0, wx0, out_v0, sem_g0, sem_o0)

        @pl.when(jnp.logical_not(par0))
        def _():
            consume(i, rows_a1, rows_b1, wy1, wx1, out_v1, sem_g1, sem_o1)

        return carry

    lax.fori_loop(0, cnt, box_body, 0)
    pltpu.make_async_copy(out_v0, out_hbm.at[pl.ds(0, NPX)], sem_o0).wait()
    pltpu.make_async_copy(out_v1, out_hbm.at[pl.ds(0, NPX)], sem_o1).wait()


def _make_ctab():
    import numpy as np
    rows = []
    for j in range(7):
        gc = np.minimum(np.arange(j * 16, j * 16 + 16), NIDX - 1)
        p = gc % NPX
        rows += [p // POOL_W, p % POOL_W, (gc < NPX).astype(np.int64), gc]
    return jnp.asarray(np.concatenate(rows), jnp.int32)


@jax.jit
def _roialign(boxes_flat, meta_flat, ctab, f2, f3, f4, f5):
    mesh = plsc.VectorSubcoreMesh(core_axis_name="c", subcore_axis_name="s",
                                  num_cores=2, num_subcores=16)
    return pl.kernel(
        _roi_body,
        out_type=jax.ShapeDtypeStruct((NBOX * NPX, C), jnp.float32),
        mesh=mesh,
        scratch_types=[
            pltpu.VMEM((BPW * 8 + 16,), jnp.float32),
            pltpu.VMEM((96,), jnp.float32),
            pltpu.VMEM((448,), jnp.int32),
            pltpu.VMEM((16,), jnp.int32),
            pltpu.VMEM((16,), jnp.int32),
            pltpu.VMEM((16,), jnp.int32),
            pltpu.VMEM((16,), jnp.int32),
            pltpu.VMEM((8, 16), jnp.float32),
            pltpu.VMEM((8, 16), jnp.float32),
            pltpu.VMEM((8, 16), jnp.float32),
            pltpu.VMEM((8, 16), jnp.float32),
            pltpu.VMEM((NIDX,), jnp.int32),
            pltpu.VMEM((NIDX,), jnp.int32),
            pltpu.VMEM((NIDX,), jnp.int32),
            pltpu.VMEM((NIDX,), jnp.int32),
            pltpu.VMEM((NIDX, C), jnp.float32),
            pltpu.VMEM((NIDX, C), jnp.float32),
            pltpu.VMEM((NIDX, C), jnp.float32),
            pltpu.VMEM((NIDX, C), jnp.float32),
            pltpu.VMEM((NPX, C), jnp.float32),
            pltpu.VMEM((NPX, C), jnp.float32),
            pltpu.SemaphoreType.DMA,
            pltpu.SemaphoreType.DMA,
            pltpu.SemaphoreType.DMA,
            pltpu.SemaphoreType.DMA,
        ],
        compiler_params=pltpu.CompilerParams(use_tc_tiling_on_sc=False,
                                             needs_layout_passes=False),
    )(boxes_flat, meta_flat, ctab, f2, f3, f4, f5)


def kernel(boxes, image_meta, p2, p3, p4, p5):
    n = boxes.shape[1]
    boxes8 = jnp.pad(boxes.reshape(-1, 4), ((0, 1026 - n), (0, 4)))
    meta_flat = jnp.pad(image_meta.reshape(-1), (0, 96 - image_meta.size))
    out = _roialign(boxes8.reshape(-1), meta_flat, _make_ctab(),
                    p2.reshape(-1, C), p3.reshape(-1, C),
                    p4.reshape(-1, C), p5.reshape(-1, C))
    return out.reshape(1, n, POOL_H, POOL_W, C)


# speedup vs baseline: 1.1084x; 1.1084x over previous
import jax
import jax.numpy as jnp
from jax import lax
from jax.experimental import pallas as pl
from jax.experimental.pallas import tpu as pltpu
from jax.experimental.pallas import tpu_sc as plsc

POOL_H = 7
POOL_W = 7
NPX = POOL_H * POOL_W
NIDX = 2 * NPX
C = 256
NW = 32
NBOX = 1000
BPW = 32
NIDXP = 104
_CHUNK_OFFS = (0, 16, 32, 48, 64, 80, 88)


def _roi_body(boxes_hbm, meta_hbm, ctab_hbm, f2, f3, f4, f5, out_hbm,
              bx_v, meta_v, ct_v, y0t, y1t, x0t, x1t,
              wy0, wx0, wy1, wx1,
              idx_a0, idx_b0, idx_a1, idx_b1, oidx, oidx2,
              rows_a0, rows_b0, rows_a1, rows_b1,
              out_v, sem_g0, sem_g1, sem_o):
    cid = lax.axis_index("c")
    sid = lax.axis_index("s")
    wid = sid * 2 + cid
    base = wid * 31 + jnp.minimum(wid, 8)
    cnt = 31 + (wid < 8).astype(jnp.int32)

    pltpu.sync_copy(boxes_hbm.at[pl.ds(base * 8, BPW * 8 + 16)], bx_v)
    pltpu.sync_copy(meta_hbm, meta_v)
    pltpu.sync_copy(ctab_hbm, ct_v)
    mv = meta_v[pl.ds(0, 16)]
    area = mv[4] * mv[5]
    th3 = jnp.float32(224.0 * 224.0 * 0.125)
    th4 = jnp.float32(224.0 * 224.0 * 0.5)
    th5 = jnp.float32(224.0 * 224.0 * 2.0)
    lanes = lax.broadcasted_iota(jnp.int32, (16,), 0)
    lanesf = lanes.astype(jnp.float32)

    def prefetch(i, idx_a, idx_b, rows_a, rows_b, wysp, wxsp, sem_g):
        bv = bx_v[pl.ds(i * 8, 16)]
        y1 = bv[0]
        x1 = bv[1]
        y2 = bv[2]
        x2 = bv[3]
        bh = y2 - y1
        bw = x2 - x1
        hw = bh * bw * area
        lvl = (2 + (hw > th3).astype(jnp.int32)
               + (hw > th4).astype(jnp.int32)
               + (hw > th5).astype(jnp.int32))
        wdim = lax.shift_right_logical(jnp.int32(256), lvl - 2)
        wm1 = wdim - 1
        wm1f = wm1.astype(jnp.float32)

        ysv = y1 * wm1f + lanesf * (bh * wm1f * (1.0 / 6.0))
        xsv = x1 * wm1f + lanesf * (bw * wm1f * (1.0 / 6.0))
        y0i = ysv.astype(jnp.int32)
        x0i = xsv.astype(jnp.int32)
        wyv = ysv - y0i.astype(jnp.float32)
        wxv = xsv - x0i.astype(jnp.float32)
        y0c = jnp.maximum(jnp.minimum(y0i, wm1), 0)
        x0c = jnp.maximum(jnp.minimum(x0i, wm1), 0)
        y0t[...] = y0c
        y1t[...] = jnp.minimum(y0c + 1, wm1)
        x0t[...] = x0c
        x1t[...] = jnp.minimum(x0c + 1, wm1)
        for k in range(POOL_H):
            wysp[k, :] = jnp.full((16,), wyv[k])
            wxsp[k, :] = jnp.full((16,), wxv[k])

        for ytab, idxref in ((y0t, idx_a), (y1t, idx_b)):
            for j, off in enumerate(_CHUNK_OFFS):
                iy = ct_v[pl.ds(j * 48, 16)]
                ix = ct_v[pl.ds(j * 48 + 16, 16)]
                is_tl = ct_v[pl.ds(j * 48 + 32, 16)] > 0
                yv = plsc.load_gather(ytab, [iy])
                xv = jnp.where(is_tl,
                               plsc.load_gather(x0t, [ix]),
                               plsc.load_gather(x1t, [ix]))
                idxv = yv * wdim + xv
                idxref[pl.ds(off, 16)] = idxv

        for level, fmap in ((2, f2), (3, f3), (4, f4), (5, f5)):
            @pl.when(lvl == level)
            def _():
                pltpu.async_copy(fmap.at[idx_a], rows_a, sem_g)
                pltpu.async_copy(fmap.at[idx_b], rows_b, sem_g)

    def consume(i, rows_a, rows_b, wysp, wxsp, sem_g):
        pltpu.make_async_copy(f2.at[idx_a0], rows_a, sem_g).wait()
        pltpu.make_async_copy(f2.at[idx_b0], rows_b, sem_g).wait()

        def iy_body(iy, c1):
            wyv = wysp[iy, :]
            omy = 1.0 - wyv

            def ix_body(ix, c2):
                wxv = wxsp[ix, :]
                omx = 1.0 - wxv
                w00 = omy * omx
                w01 = omy * wxv
                w10 = wyv * omx
                w11 = wyv * wxv
                p = iy * POOL_W + ix
                q = p + NPX
                for ck in range(C // 16):
                    sl = pl.ds(ck * 16, 16)
                    out_v[p, sl] = (rows_a[p, sl] * w00 + rows_a[q, sl] * w01
                                    + rows_b[p, sl] * w10 + rows_b[q, sl] * w11)
                return c2
            lax.fori_loop(0, POOL_W, ix_body, 0)
            return c1
        lax.fori_loop(0, POOL_H, iy_body, 0)

        for ck in range(C // 16):
            sl = pl.ds(ck * 16, 16)
            v48 = out_v[48, sl]
            for r in range(49, 64):
                out_v[r, sl] = v48
        obase = (base + i) * NPX
        for off in (0, 16, 32):
            oidx[pl.ds(off, 16)] = obase + (lanes + off)
        oidx2[...] = jnp.full((16,), obase + (NPX - 1), jnp.int32)
        pltpu.async_copy(out_v.at[pl.ds(0, 48)], out_hbm.at[oidx], sem_o)
        pltpu.async_copy(out_v.at[pl.ds(48, 16)], out_hbm.at[oidx2], sem_o)
        pltpu.make_async_copy(out_v.at[pl.ds(0, 48)], out_hbm.at[oidx],
                              sem_o).wait()
        pltpu.make_async_copy(out_v.at[pl.ds(48, 16)], out_hbm.at[oidx2],
                              sem_o).wait()

    prefetch(0, idx_a0, idx_b0, rows_a0, rows_b0, wy0, wx0, sem_g0)

    def box_body(i, carry):
        par0 = (i & 1) == 0
        nxt = i + 1

        @pl.when(nxt < cnt)
        def _():
            @pl.when(par0)
            def _():
                prefetch(nxt, idx_a1, idx_b1, rows_a1, rows_b1,
                         wy1, wx1, sem_g1)

            @pl.when(jnp.logical_not(par0))
            def _():
                prefetch(nxt, idx_a0, idx_b0, rows_a0, rows_b0,
                         wy0, wx0, sem_g0)

        @pl.when(par0)
        def _():
            consume(i, rows_a0, rows_b0, wy0, wx0, sem_g0)

        @pl.when(jnp.logical_not(par0))
        def _():
            consume(i, rows_a1, rows_b1, wy1, wx1, sem_g1)

        return carry

    lax.fori_loop(0, cnt, box_body, 0)


def _make_ctab():
    import numpy as np
    rows = []
    for off in _CHUNK_OFFS:
        g = np.minimum(np.arange(off, off + 16), NIDX - 1)
        p = g % NPX
        rows += [p // POOL_W, p % POOL_W, (g < NPX).astype(np.int64)]
    return jnp.asarray(np.concatenate(rows), jnp.int32)


@jax.jit
def _roialign(boxes_flat, meta_flat, ctab, f2, f3, f4, f5):
    mesh = plsc.VectorSubcoreMesh(core_axis_name="c", subcore_axis_name="s",
                                  num_cores=2, num_subcores=16)
    return pl.kernel(
        _roi_body,
        out_type=jax.ShapeDtypeStruct((NBOX * NPX, C), jnp.float32),
        mesh=mesh,
        scratch_types=[
            pltpu.VMEM((BPW * 8 + 16,), jnp.float32),
            pltpu.VMEM((96,), jnp.float32),
            pltpu.VMEM((336,), jnp.int32),
            pltpu.VMEM((16,), jnp.int32),
            pltpu.VMEM((16,), jnp.int32),
            pltpu.VMEM((16,), jnp.int32),
            pltpu.VMEM((16,), jnp.int32),
            pltpu.VMEM((8, 16), jnp.float32),
            pltpu.VMEM((8, 16), jnp.float32),
            pltpu.VMEM((8, 16), jnp.float32),
            pltpu.VMEM((8, 16), jnp.float32),
            pltpu.VMEM((NIDXP,), jnp.int32),
            pltpu.VMEM((NIDXP,), jnp.int32),
            pltpu.VMEM((NIDXP,), jnp.int32),
            pltpu.VMEM((NIDXP,), jnp.int32),
            pltpu.VMEM((48,), jnp.int32),
            pltpu.VMEM((16,), jnp.int32),
            pltpu.VMEM((NIDXP, C), jnp.float32),
            pltpu.VMEM((NIDXP, C), jnp.float32),
            pltpu.VMEM((NIDXP, C), jnp.float32),
            pltpu.VMEM((NIDXP, C), jnp.float32),
            pltpu.VMEM((64, C), jnp.float32),
            pltpu.SemaphoreType.DMA,
            pltpu.SemaphoreType.DMA,
            pltpu.SemaphoreType.DMA,
        ],
        compiler_params=pltpu.CompilerParams(use_tc_tiling_on_sc=True,
                                             needs_layout_passes=False),
    )(boxes_flat, meta_flat, ctab, f2, f3, f4, f5)


def kernel(boxes, image_meta, p2, p3, p4, p5):
    n = boxes.shape[1]
    boxes8 = jnp.pad(boxes.reshape(-1, 4), ((0, 1026 - n), (0, 4)))
    meta_flat = jnp.pad(image_meta.reshape(-1), (0, 96 - image_meta.size))
    out = _roialign(boxes8.reshape(-1), meta_flat, _make_ctab(),
                    p2.reshape(-1, C), p3.reshape(-1, C),
                    p4.reshape(-1, C), p5.reshape(-1, C))
    return out.reshape(1, n, POOL_H, POOL_W, C)


# speedup vs baseline: 1.2334x; 1.1128x over previous
import jax
import jax.numpy as jnp
from jax import lax
from jax.experimental import pallas as pl
from jax.experimental.pallas import tpu as pltpu
from jax.experimental.pallas import tpu_sc as plsc

POOL_H = 7
POOL_W = 7
NPX = POOL_H * POOL_W
NIDX = 2 * NPX
C = 256
NW = 32
NBOX = 1000
BPW = 32
NIDXP = 104
_CHUNK_OFFS = (0, 16, 32, 48, 64, 80, 88)


def _roi_body(boxes_hbm, meta_hbm, ctab_hbm, f2, f3, f4, f5, out_hbm,
              bx_v, meta_v, ct_v, y0t, y1t, x0t, x1t,
              wy0, wx0, wy1, wx1,
              idx_a0, idx_b0, idx_a1, idx_b1,
              rows_a0, rows_b0, rows_a1, rows_b1,
              out_v, sem_g0, sem_g1, sem_o):
    cid = lax.axis_index("c")
    sid = lax.axis_index("s")
    wid = sid * 2 + cid
    base = wid * 31 + jnp.minimum(wid, 8)
    cnt = 31 + (wid < 8).astype(jnp.int32)

    pltpu.sync_copy(boxes_hbm.at[pl.ds(base * 8, BPW * 8 + 16)], bx_v)
    pltpu.sync_copy(meta_hbm, meta_v)
    pltpu.sync_copy(ctab_hbm, ct_v)
    mv = meta_v[pl.ds(0, 16)]
    area = mv[4] * mv[5]
    th3 = jnp.float32(224.0 * 224.0 * 0.125)
    th4 = jnp.float32(224.0 * 224.0 * 0.5)
    th5 = jnp.float32(224.0 * 224.0 * 2.0)
    lanes = lax.broadcasted_iota(jnp.int32, (16,), 0)
    lanesf = lanes.astype(jnp.float32)

    def prefetch(i, idx_a, idx_b, rows_a, rows_b, wysp, wxsp, sem_g):
        bv = bx_v[pl.ds(i * 8, 16)]
        y1 = bv[0]
        x1 = bv[1]
        y2 = bv[2]
        x2 = bv[3]
        bh = y2 - y1
        bw = x2 - x1
        hw = bh * bw * area
        lvl = (2 + (hw > th3).astype(jnp.int32)
               + (hw > th4).astype(jnp.int32)
               + (hw > th5).astype(jnp.int32))
        wdim = lax.shift_right_logical(jnp.int32(256), lvl - 2)
        wm1 = wdim - 1
        wm1f = wm1.astype(jnp.float32)

        ysv = y1 * wm1f + lanesf * (bh * wm1f * (1.0 / 6.0))
        xsv = x1 * wm1f + lanesf * (bw * wm1f * (1.0 / 6.0))
        y0i = ysv.astype(jnp.int32)
        x0i = xsv.astype(jnp.int32)
        wyv = ysv - y0i.astype(jnp.float32)
        wxv = xsv - x0i.astype(jnp.float32)
        y0c = jnp.maximum(jnp.minimum(y0i, wm1), 0)
        x0c = jnp.maximum(jnp.minimum(x0i, wm1), 0)
        y0t[...] = y0c
        y1t[...] = jnp.minimum(y0c + 1, wm1)
        x0t[...] = x0c
        x1t[...] = jnp.minimum(x0c + 1, wm1)
        for k in range(POOL_H):
            wysp[k, :] = jnp.full((16,), wyv[k])
            wxsp[k, :] = jnp.full((16,), wxv[k])

        for ytab, idxref in ((y0t, idx_a), (y1t, idx_b)):
            for j, off in enumerate(_CHUNK_OFFS):
                iy = ct_v[pl.ds(j * 48, 16)]
                ix = ct_v[pl.ds(j * 48 + 16, 16)]
                is_tl = ct_v[pl.ds(j * 48 + 32, 16)] > 0
                yv = plsc.load_gather(ytab, [iy])
                xv = jnp.where(is_tl,
                               plsc.load_gather(x0t, [ix]),
                               plsc.load_gather(x1t, [ix]))
                idxv = yv * wdim + xv
                idxref[pl.ds(off, 16)] = idxv

        for level, fmap in ((2, f2), (3, f3), (4, f4), (5, f5)):
            @pl.when(lvl == level)
            def _():
                pltpu.async_copy(fmap.at[idx_a], rows_a, sem_g)
                pltpu.async_copy(fmap.at[idx_b], rows_b, sem_g)

    def consume(i, rows_a, rows_b, wysp, wxsp, sem_g):
        pltpu.make_async_copy(f2.at[idx_a0], rows_a, sem_g).wait()
        pltpu.make_async_copy(f2.at[idx_b0], rows_b, sem_g).wait()

        def iy_body(iy, c1):
            wyv = wysp[iy, :]
            omy = 1.0 - wyv

            def ix_body(ix, c2):
                wxv = wxsp[ix, :]
                omx = 1.0 - wxv
                w00 = omy * omx
                w01 = omy * wxv
                w10 = wyv * omx
                w11 = wyv * wxv
                p = iy * POOL_W + ix
                q = p + NPX
                for ck in range(C // 16):
                    sl = pl.ds(ck * 16, 16)
                    out_v[iy, ix, sl] = (rows_a[p, sl] * w00
                                         + rows_a[q, sl] * w01
                                         + rows_b[p, sl] * w10
                                         + rows_b[q, sl] * w11)
                return c2
            lax.fori_loop(0, POOL_W, ix_body, 0)
            return c1
        lax.fori_loop(0, POOL_H, iy_body, 0)

        pltpu.async_copy(out_v, out_hbm.at[base + i], sem_o)
        pltpu.make_async_copy(out_v, out_hbm.at[base + i], sem_o).wait()

    prefetch(0, idx_a0, idx_b0, rows_a0, rows_b0, wy0, wx0, sem_g0)

    def box_body(i, carry):
        par0 = (i & 1) == 0
        nxt = i + 1

        @pl.when(nxt < cnt)
        def _():
            @pl.when(par0)
            def _():
                prefetch(nxt, idx_a1, idx_b1, rows_a1, rows_b1,
                         wy1, wx1, sem_g1)

            @pl.when(jnp.logical_not(par0))
            def _():
                prefetch(nxt, idx_a0, idx_b0, rows_a0, rows_b0,
                         wy0, wx0, sem_g0)

        @pl.when(par0)
        def _():
            consume(i, rows_a0, rows_b0, wy0, wx0, sem_g0)

        @pl.when(jnp.logical_not(par0))
        def _():
            consume(i, rows_a1, rows_b1, wy1, wx1, sem_g1)

        return carry

    lax.fori_loop(0, cnt, box_body, 0)


def _make_ctab():
    import numpy as np
    rows = []
    for off in _CHUNK_OFFS:
        g = np.minimum(np.arange(off, off + 16), NIDX - 1)
        p = g % NPX
        rows += [p // POOL_W, p % POOL_W, (g < NPX).astype(np.int64)]
    return jnp.asarray(np.concatenate(rows), jnp.int32)


@jax.jit
def _roialign(boxes_flat, meta_flat, ctab, f2, f3, f4, f5):
    mesh = plsc.VectorSubcoreMesh(core_axis_name="c", subcore_axis_name="s",
                                  num_cores=2, num_subcores=16)
    return pl.kernel(
        _roi_body,
        out_type=jax.ShapeDtypeStruct((NBOX, POOL_H, POOL_W, C), jnp.float32),
        mesh=mesh,
        scratch_types=[
            pltpu.VMEM((BPW * 8 + 16,), jnp.float32),
            pltpu.VMEM((96,), jnp.float32),
            pltpu.VMEM((336,), jnp.int32),
            pltpu.VMEM((16,), jnp.int32),
            pltpu.VMEM((16,), jnp.int32),
            pltpu.VMEM((16,), jnp.int32),
            pltpu.VMEM((16,), jnp.int32),
            pltpu.VMEM((8, 16), jnp.float32),
            pltpu.VMEM((8, 16), jnp.float32),
            pltpu.VMEM((8, 16), jnp.float32),
            pltpu.VMEM((8, 16), jnp.float32),
            pltpu.VMEM((NIDXP,), jnp.int32),
            pltpu.VMEM((NIDXP,), jnp.int32),
            pltpu.VMEM((NIDXP,), jnp.int32),
            pltpu.VMEM((NIDXP,), jnp.int32),
            pltpu.VMEM((NIDXP, C), jnp.float32),
            pltpu.VMEM((NIDXP, C), jnp.float32),
            pltpu.VMEM((NIDXP, C), jnp.float32),
            pltpu.VMEM((NIDXP, C), jnp.float32),
            pltpu.VMEM((POOL_H, POOL_W, C), jnp.float32),
            pltpu.SemaphoreType.DMA,
            pltpu.SemaphoreType.DMA,
            pltpu.SemaphoreType.DMA,
        ],
        compiler_params=pltpu.CompilerParams(use_tc_tiling_on_sc=True,
                                             needs_layout_passes=False),
    )(boxes_flat, meta_flat, ctab, f2, f3, f4, f5)


def kernel(boxes, image_meta, p2, p3, p4, p5):
    n = boxes.shape[1]
    boxes8 = jnp.pad(boxes.reshape(-1, 4), ((0, 1026 - n), (0, 4)))
    meta_flat = jnp.pad(image_meta.reshape(-1), (0, 96 - image_meta.size))
    out = _roialign(boxes8.reshape(-1), meta_flat, _make_ctab(),
                    p2.reshape(-1, C), p3.reshape(-1, C),
                    p4.reshape(-1, C), p5.reshape(-1, C))
    return out.reshape(1, n, POOL_H, POOL_W, C)


# speedup vs baseline: 2.1155x; 1.7152x over previous
import jax
import jax.numpy as jnp
from jax import lax
from jax.experimental import pallas as pl
from jax.experimental.pallas import tpu as pltpu
from jax.experimental.pallas import tpu_sc as plsc

POOL_H = 7
POOL_W = 7
NPX = POOL_H * POOL_W
NIDX = 2 * NPX
C = 256
NW = 32
NBOX = 1000
BPW = 32
NIDXP = 104
_CHUNK_OFFS = (0, 16, 32, 48, 64, 80, 88)


def _roi_body(boxes_hbm, meta_hbm, ctab_hbm, f2, f3, f4, f5, out_hbm,
              bx_v, meta_v, ct_v, y0t, y1t, x0t, x1t,
              wy0, wx0, wy1, wx1,
              idx_a0, idx_b0, idx_a1, idx_b1, oidx, oidx2,
              rows_a0, rows_b0, rows_a1, rows_b1,
              out_v, sem_g0, sem_g1, sem_o):
    cid = lax.axis_index("c")
    sid = lax.axis_index("s")
    wid = sid * 2 + cid
    base = wid * 31 + jnp.minimum(wid, 8)
    cnt = 31 + (wid < 8).astype(jnp.int32)

    pltpu.sync_copy(boxes_hbm.at[pl.ds(base * 8, BPW * 8 + 16)], bx_v)
    pltpu.sync_copy(meta_hbm, meta_v)
    pltpu.sync_copy(ctab_hbm, ct_v)
    mv = meta_v[pl.ds(0, 16)]
    area = mv[4] * mv[5]
    th3 = jnp.float32(224.0 * 224.0 * 0.125)
    th4 = jnp.float32(224.0 * 224.0 * 0.5)
    th5 = jnp.float32(224.0 * 224.0 * 2.0)
    lanes = lax.broadcasted_iota(jnp.int32, (16,), 0)
    lanesf = lanes.astype(jnp.float32)

    def prefetch(i, idx_a, idx_b, rows_a, rows_b, wysp, wxsp, sem_g):
        bv = bx_v[pl.ds(i * 8, 16)]
        y1 = bv[0]
        x1 = bv[1]
        y2 = bv[2]
        x2 = bv[3]
        bh = y2 - y1
        bw = x2 - x1
        hw = bh * bw * area
        lvl = (2 + (hw > th3).astype(jnp.int32)
               + (hw > th4).astype(jnp.int32)
               + (hw > th5).astype(jnp.int32))
        wdim = lax.shift_right_logical(jnp.int32(256), lvl - 2)
        wm1 = wdim - 1
        wm1f = wm1.astype(jnp.float32)

        ysv = y1 * wm1f + lanesf * (bh * wm1f * (1.0 / 6.0))
        xsv = x1 * wm1f + lanesf * (bw * wm1f * (1.0 / 6.0))
        y0i = ysv.astype(jnp.int32)
        x0i = xsv.astype(jnp.int32)
        wyv = ysv - y0i.astype(jnp.float32)
        wxv = xsv - x0i.astype(jnp.float32)
        y0c = jnp.maximum(jnp.minimum(y0i, wm1), 0)
        x0c = jnp.maximum(jnp.minimum(x0i, wm1), 0)
        y0t[...] = y0c
        y1t[...] = jnp.minimum(y0c + 1, wm1)
        x0t[...] = x0c
        x1t[...] = jnp.minimum(x0c + 1, wm1)
        for k in range(POOL_H):
            wysp[k, :] = jnp.full((16,), wyv[k])
            wxsp[k, :] = jnp.full((16,), wxv[k])

        for ytab, idxref in ((y0t, idx_a), (y1t, idx_b)):
            for j, off in enumerate(_CHUNK_OFFS):
                iy = ct_v[pl.ds(j * 48, 16)]
                ix = ct_v[pl.ds(j * 48 + 16, 16)]
                is_tl = ct_v[pl.ds(j * 48 + 32, 16)] > 0
                yv = plsc.load_gather(ytab, [iy])
                xv = jnp.where(is_tl,
                               plsc.load_gather(x0t, [ix]),
                               plsc.load_gather(x1t, [ix]))
                idxv = yv * wdim + xv
                idxref[pl.ds(off, 16)] = idxv

        for level, fmap in ((2, f2), (3, f3), (4, f4), (5, f5)):
            @pl.when(lvl == level)
            def _():
                pltpu.async_copy(fmap.at[idx_a], rows_a, sem_g)
                pltpu.async_copy(fmap.at[idx_b], rows_b, sem_g)

    def consume(i, rows_a, rows_b, wysp, wxsp, sem_g):
        pltpu.make_async_copy(f2.at[idx_a0], rows_a, sem_g).wait()
        pltpu.make_async_copy(f2.at[idx_b0], rows_b, sem_g).wait()

        def iy_body(iy, c1):
            wyv = wysp[iy, :]
            omy = 1.0 - wyv

            def ix_body(ix, c2):
                wxv = wxsp[ix, :]
                omx = 1.0 - wxv
                w00 = omy * omx
                w01 = omy * wxv
                w10 = wyv * omx
                w11 = wyv * wxv
                p = iy * POOL_W + ix
                q = p + NPX
                for ck in range(C // 16):
                    sl = pl.ds(ck * 16, 16)
                    out_v[p, sl] = (rows_a[p, sl] * w00 + rows_a[q, sl] * w01
                                    + rows_b[p, sl] * w10 + rows_b[q, sl] * w11)
                return c2
            lax.fori_loop(0, POOL_W, ix_body, 0)
            return c1
        lax.fori_loop(0, POOL_H, iy_body, 0)

        for ck in range(C // 16):
            sl = pl.ds(ck * 16, 16)
            v48 = out_v[48, sl]
            for r in range(49, 64):
                out_v[r, sl] = v48
        b = base + i
        for off in (0, 16, 32):
            oidx[pl.ds(off, 16)] = b + (lanes + off) * NBOX
        oidx2[...] = jnp.full((16,), b + (NPX - 1) * NBOX, jnp.int32)
        pltpu.async_copy(out_v.at[pl.ds(0, 48)], out_hbm.at[oidx], sem_o)
        pltpu.async_copy(out_v.at[pl.ds(48, 16)], out_hbm.at[oidx2], sem_o)
        pltpu.make_async_copy(out_v.at[pl.ds(0, 48)], out_hbm.at[oidx],
                              sem_o).wait()
        pltpu.make_async_copy(out_v.at[pl.ds(48, 16)], out_hbm.at[oidx2],
                              sem_o).wait()

    prefetch(0, idx_a0, idx_b0, rows_a0, rows_b0, wy0, wx0, sem_g0)

    def box_body(i, carry):
        par0 = (i & 1) == 0
        nxt = i + 1

        @pl.when(nxt < cnt)
        def _():
            @pl.when(par0)
            def _():
                prefetch(nxt, idx_a1, idx_b1, rows_a1, rows_b1,
                         wy1, wx1, sem_g1)

            @pl.when(jnp.logical_not(par0))
            def _():
                prefetch(nxt, idx_a0, idx_b0, rows_a0, rows_b0,
                         wy0, wx0, sem_g0)

        @pl.when(par0)
        def _():
            consume(i, rows_a0, rows_b0, wy0, wx0, sem_g0)

        @pl.when(jnp.logical_not(par0))
        def _():
            consume(i, rows_a1, rows_b1, wy1, wx1, sem_g1)

        return carry

    lax.fori_loop(0, cnt, box_body, 0)


def _make_ctab():
    import numpy as np
    rows = []
    for off in _CHUNK_OFFS:
        g = np.minimum(np.arange(off, off + 16), NIDX - 1)
        p = g % NPX
        rows += [p // POOL_W, p % POOL_W, (g < NPX).astype(np.int64)]
    return jnp.asarray(np.concatenate(rows), jnp.int32)


@jax.jit
def _roialign(boxes_flat, meta_flat, ctab, f2, f3, f4, f5):
    mesh = plsc.VectorSubcoreMesh(core_axis_name="c", subcore_axis_name="s",
                                  num_cores=2, num_subcores=16)
    return pl.kernel(
        _roi_body,
        out_type=jax.ShapeDtypeStruct((NPX * NBOX, C), jnp.float32),
        mesh=mesh,
        scratch_types=[
            pltpu.VMEM((BPW * 8 + 16,), jnp.float32),
            pltpu.VMEM((96,), jnp.float32),
            pltpu.VMEM((336,), jnp.int32),
            pltpu.VMEM((16,), jnp.int32),
            pltpu.VMEM((16,), jnp.int32),
            pltpu.VMEM((16,), jnp.int32),
            pltpu.VMEM((16,), jnp.int32),
            pltpu.VMEM((8, 16), jnp.float32),
            pltpu.VMEM((8, 16), jnp.float32),
            pltpu.VMEM((8, 16), jnp.float32),
            pltpu.VMEM((8, 16), jnp.float32),
            pltpu.VMEM((NIDXP,), jnp.int32),
            pltpu.VMEM((NIDXP,), jnp.int32),
            pltpu.VMEM((NIDXP,), jnp.int32),
            pltpu.VMEM((NIDXP,), jnp.int32),
            pltpu.VMEM((48,), jnp.int32),
            pltpu.VMEM((16,), jnp.int32),
            pltpu.VMEM((NIDXP, C), jnp.float32),
            pltpu.VMEM((NIDXP, C), jnp.float32),
            pltpu.VMEM((NIDXP, C), jnp.float32),
            pltpu.VMEM((NIDXP, C), jnp.float32),
            pltpu.VMEM((64, C), jnp.float32),
            pltpu.SemaphoreType.DMA,
            pltpu.SemaphoreType.DMA,
            pltpu.SemaphoreType.DMA,
        ],
        compiler_params=pltpu.CompilerParams(use_tc_tiling_on_sc=True,
                                             needs_layout_passes=False),
    )(boxes_flat, meta_flat, ctab, f2, f3, f4, f5)


def kernel(boxes, image_meta, p2, p3, p4, p5):
    n = boxes.shape[1]
    boxes8 = jnp.pad(boxes.reshape(-1, 4), ((0, 1026 - n), (0, 4)))
    meta_flat = jnp.pad(image_meta.reshape(-1), (0, 96 - image_meta.size))
    out = _roialign(boxes8.reshape(-1), meta_flat, _make_ctab(),
                    p2.reshape(-1, C), p3.reshape(-1, C),
                    p4.reshape(-1, C), p5.reshape(-1, C))
    out5 = out.reshape(1, POOL_H, POOL_W, n, C)
    return jnp.transpose(out5, (0, 3, 1, 2, 4))


# speedup vs baseline: 2.1975x; 1.0387x over previous
import jax
import jax.numpy as jnp
from jax import lax
from jax.experimental import pallas as pl
from jax.experimental.pallas import tpu as pltpu
from jax.experimental.pallas import tpu_sc as plsc

POOL_H = 7
POOL_W = 7
NPX = POOL_H * POOL_W
NIDX = 2 * NPX
C = 256
NW = 32
NBOX = 1000
BPW = 32
NIDXP = 104
_CHUNK_OFFS = (0, 16, 32, 48, 64, 80, 88)


def _roi_body(boxes_hbm, meta_hbm, ctab_hbm, f2, f3, f4, f5, out_hbm,
              bx_v, meta_v, ct_v, y0t, y1t, x0t, x1t,
              wy0, wx0, wy1, wx1,
              idx_a0, idx_b0, idx_a1, idx_b1, oidx, oidx2,
              rows_a0, rows_b0, rows_a1, rows_b1,
              out_v, sem_g0, sem_g1, sem_o):
    cid = lax.axis_index("c")
    sid = lax.axis_index("s")
    wid = sid * 2 + cid
    base = wid * 31 + jnp.minimum(wid, 8)
    cnt = 31 + (wid < 8).astype(jnp.int32)

    pltpu.sync_copy(boxes_hbm.at[pl.ds(base * 8, BPW * 8 + 16)], bx_v)
    pltpu.sync_copy(meta_hbm, meta_v)
    pltpu.sync_copy(ctab_hbm, ct_v)
    mv = meta_v[pl.ds(0, 16)]
    area = mv[4] * mv[5]
    th3 = jnp.float32(224.0 * 224.0 * 0.125)
    th4 = jnp.float32(224.0 * 224.0 * 0.5)
    th5 = jnp.float32(224.0 * 224.0 * 2.0)
    lanes = lax.broadcasted_iota(jnp.int32, (16,), 0)
    lanesf = lanes.astype(jnp.float32)

    def prefetch(i, idx_a, idx_b, rows_a, rows_b, wysp, wxsp, sem_g):
        bv = bx_v[pl.ds(i * 8, 16)]
        y1 = bv[0]
        x1 = bv[1]
        y2 = bv[2]
        x2 = bv[3]
        bh = y2 - y1
        bw = x2 - x1
        hw = bh * bw * area
        lvl = (2 + (hw > th3).astype(jnp.int32)
               + (hw > th4).astype(jnp.int32)
               + (hw > th5).astype(jnp.int32))
        wdim = lax.shift_right_logical(jnp.int32(256), lvl - 2)
        wm1 = wdim - 1
        wm1f = wm1.astype(jnp.float32)

        ysv = y1 * wm1f + lanesf * (bh * wm1f * (1.0 / 6.0))
        xsv = x1 * wm1f + lanesf * (bw * wm1f * (1.0 / 6.0))
        y0i = ysv.astype(jnp.int32)
        x0i = xsv.astype(jnp.int32)
        wyv = ysv - y0i.astype(jnp.float32)
        wxv = xsv - x0i.astype(jnp.float32)
        y0c = jnp.maximum(jnp.minimum(y0i, wm1), 0)
        x0c = jnp.maximum(jnp.minimum(x0i, wm1), 0)
        y0t[...] = y0c
        y1t[...] = jnp.minimum(y0c + 1, wm1)
        x0t[...] = x0c
        x1t[...] = jnp.minimum(x0c + 1, wm1)
        for k in range(POOL_H):
            wysp[k, :] = jnp.full((16,), wyv[k])
            wxsp[k, :] = jnp.full((16,), wxv[k])

        for ytab, idxref in ((y0t, idx_a), (y1t, idx_b)):
            for j, off in enumerate(_CHUNK_OFFS):
                iy = ct_v[pl.ds(j * 48, 16)]
                ix = ct_v[pl.ds(j * 48 + 16, 16)]
                is_tl = ct_v[pl.ds(j * 48 + 32, 16)] > 0
                yv = plsc.load_gather(ytab, [iy])
                xv = jnp.where(is_tl,
                               plsc.load_gather(x0t, [ix]),
                               plsc.load_gather(x1t, [ix]))
                idxv = yv * wdim + xv
                idxref[pl.ds(off, 16)] = idxv

        for level, fmap in ((2, f2), (3, f3), (4, f4), (5, f5)):
            @pl.when(lvl == level)
            def _():
                pltpu.async_copy(fmap.at[idx_a], rows_a, sem_g)
                pltpu.async_copy(fmap.at[idx_b], rows_b, sem_g)

    def consume(i, rows_a, rows_b, wysp, wxsp, sem_g):
        pltpu.make_async_copy(f2.at[idx_a0], rows_a, sem_g).wait()
        pltpu.make_async_copy(f2.at[idx_b0], rows_b, sem_g).wait()

        @pl.when(i >= 1)
        def _():
            pltpu.make_async_copy(out_v.at[pl.ds(0, 48)], out_hbm.at[oidx],
                                  sem_o).wait()
            pltpu.make_async_copy(out_v.at[pl.ds(48, 16)], out_hbm.at[oidx2],
                                  sem_o).wait()

        def iy_body(iy, c1):
            wyv = wysp[iy, :]
            omy = 1.0 - wyv

            def ix_body(ix, c2):
                wxv = wxsp[ix, :]
                omx = 1.0 - wxv
                w00 = omy * omx
                w01 = omy * wxv
                w10 = wyv * omx
                w11 = wyv * wxv
                p = iy * POOL_W + ix
                q = p + NPX
                for ck in range(C // 16):
                    sl = pl.ds(ck * 16, 16)
                    out_v[p, sl] = (rows_a[p, sl] * w00 + rows_a[q, sl] * w01
                                    + rows_b[p, sl] * w10 + rows_b[q, sl] * w11)
                return c2
            lax.fori_loop(0, POOL_W, ix_body, 0)
            return c1
        lax.fori_loop(0, POOL_H, iy_body, 0)

        for ck in range(C // 16):
            sl = pl.ds(ck * 16, 16)
            v48 = out_v[48, sl]
            for r in range(49, 64):
                out_v[r, sl] = v48
        b = base + i
        for off in (0, 16, 32):
            oidx[pl.ds(off, 16)] = b + (lanes + off) * NBOX
        oidx2[...] = jnp.full((16,), b + (NPX - 1) * NBOX, jnp.int32)
        pltpu.async_copy(out_v.at[pl.ds(0, 48)], out_hbm.at[oidx], sem_o)
        pltpu.async_copy(out_v.at[pl.ds(48, 16)], out_hbm.at[oidx2], sem_o)

    prefetch(0, idx_a0, idx_b0, rows_a0, rows_b0, wy0, wx0, sem_g0)

    def box_body(i, carry):
        par0 = (i & 1) == 0
        nxt = i + 1

        @pl.when(nxt < cnt)
        def _():
            @pl.when(par0)
            def _():
                prefetch(nxt, idx_a1, idx_b1, rows_a1, rows_b1,
                         wy1, wx1, sem_g1)

            @pl.when(jnp.logical_not(par0))
            def _():
                prefetch(nxt, idx_a0, idx_b0, rows_a0, rows_b0,
                         wy0, wx0, sem_g0)

        @pl.when(par0)
        def _():
            consume(i, rows_a0, rows_b0, wy0, wx0, sem_g0)

        @pl.when(jnp.logical_not(par0))
        def _():
            consume(i, rows_a1, rows_b1, wy1, wx1, sem_g1)

        return carry

    lax.fori_loop(0, cnt, box_body, 0)
    pltpu.make_async_copy(out_v.at[pl.ds(0, 48)], out_hbm.at[oidx],
                          sem_o).wait()
    pltpu.make_async_copy(out_v.at[pl.ds(48, 16)], out_hbm.at[oidx2],
                          sem_o).wait()


def _make_ctab():
    import numpy as np
    rows = []
    for off in _CHUNK_OFFS:
        g = np.minimum(np.arange(off, off + 16), NIDX - 1)
        p = g % NPX
        rows += [p // POOL_W, p % POOL_W, (g < NPX).astype(np.int64)]
    return jnp.asarray(np.concatenate(rows), jnp.int32)


@jax.jit
def _roialign(boxes_flat, meta_flat, ctab, f2, f3, f4, f5):
    mesh = plsc.VectorSubcoreMesh(core_axis_name="c", subcore_axis_name="s",
                                  num_cores=2, num_subcores=16)
    return pl.kernel(
        _roi_body,
        out_type=jax.ShapeDtypeStruct((NPX * NBOX, C), jnp.float32),
        mesh=mesh,
        scratch_types=[
            pltpu.VMEM((BPW * 8 + 16,), jnp.float32),
            pltpu.VMEM((96,), jnp.float32),
            pltpu.VMEM((336,), jnp.int32),
            pltpu.VMEM((16,), jnp.int32),
            pltpu.VMEM((16,), jnp.int32),
            pltpu.VMEM((16,), jnp.int32),
            pltpu.VMEM((16,), jnp.int32),
            pltpu.VMEM((8, 16), jnp.float32),
            pltpu.VMEM((8, 16), jnp.float32),
            pltpu.VMEM((8, 16), jnp.float32),
            pltpu.VMEM((8, 16), jnp.float32),
            pltpu.VMEM((NIDXP,), jnp.int32),
            pltpu.VMEM((NIDXP,), jnp.int32),
            pltpu.VMEM((NIDXP,), jnp.int32),
            pltpu.VMEM((NIDXP,), jnp.int32),
            pltpu.VMEM((48,), jnp.int32),
            pltpu.VMEM((16,), jnp.int32),
            pltpu.VMEM((NIDXP, C), jnp.float32),
            pltpu.VMEM((NIDXP, C), jnp.float32),
            pltpu.VMEM((NIDXP, C), jnp.float32),
            pltpu.VMEM((NIDXP, C), jnp.float32),
            pltpu.VMEM((64, C), jnp.float32),
            pltpu.SemaphoreType.DMA,
            pltpu.SemaphoreType.DMA,
            pltpu.SemaphoreType.DMA,
        ],
        compiler_params=pltpu.CompilerParams(use_tc_tiling_on_sc=True,
                                             needs_layout_passes=False),
    )(boxes_flat, meta_flat, ctab, f2, f3, f4, f5)


def kernel(boxes, image_meta, p2, p3, p4, p5):
    n = boxes.shape[1]
    boxes8 = jnp.pad(boxes.reshape(-1, 4), ((0, 1026 - n), (0, 4)))
    meta_flat = jnp.pad(image_meta.reshape(-1), (0, 96 - image_meta.size))
    out = _roialign(boxes8.reshape(-1), meta_flat, _make_ctab(),
                    p2.reshape(-1, C), p3.reshape(-1, C),
                    p4.reshape(-1, C), p5.reshape(-1, C))
    out5 = out.reshape(1, POOL_H, POOL_W, n, C)
    return jnp.transpose(out5, (0, 3, 1, 2, 4))


# speedup vs baseline: 3.3930x; 1.5440x over previous
import jax
import jax.numpy as jnp
from jax import lax
from jax.experimental import pallas as pl
from jax.experimental.pallas import tpu as pltpu
from jax.experimental.pallas import tpu_sc as plsc

POOL_H = 7
POOL_W = 7
NPX = POOL_H * POOL_W
NIDX = 2 * NPX
C = 256
NW = 32
NBOX = 1000
BPW = 32
NIDXP = 104
_CHUNK_OFFS = (0, 16, 32, 48, 64, 80, 88)


def _roi_body(boxes_hbm, meta_hbm, ctab_hbm, f2, f3, f4, f5, out_hbm,
              bx_v, meta_v, ct_v, y0t, y1t, x0t, x1t,
              wy0, wx0, wy1, wx1,
              idx_a0, idx_b0, idx_a1, idx_b1, oidx, oidx2,
              rows_a0, rows_b0, rows_a1, rows_b1,
              out_v, sem_g0, sem_g1, sem_o):
    cid = lax.axis_index("c")
    sid = lax.axis_index("s")
    wid = sid * 2 + cid
    base = wid * 31 + jnp.minimum(wid, 8)
    cnt = 31 + (wid < 8).astype(jnp.int32)

    pltpu.sync_copy(boxes_hbm.at[pl.ds(base * 8, BPW * 8 + 16)], bx_v)
    pltpu.sync_copy(meta_hbm, meta_v)
    pltpu.sync_copy(ctab_hbm, ct_v)
    mv = meta_v[pl.ds(0, 16)]
    area = mv[4] * mv[5]
    th3 = jnp.float32(224.0 * 224.0 * 0.125)
    th4 = jnp.float32(224.0 * 224.0 * 0.5)
    th5 = jnp.float32(224.0 * 224.0 * 2.0)
    lanes = lax.broadcasted_iota(jnp.int32, (16,), 0)
    lanesf = lanes.astype(jnp.float32)

    def prefetch(i, idx_a, idx_b, rows_a, rows_b, wysp, wxsp, sem_g):
        bv = bx_v[pl.ds(i * 8, 16)]
        y1 = bv[0]
        x1 = bv[1]
        y2 = bv[2]
        x2 = bv[3]
        bh = y2 - y1
        bw = x2 - x1
        hw = bh * bw * area
        lvl = (2 + (hw > th3).astype(jnp.int32)
               + (hw > th4).astype(jnp.int32)
               + (hw > th5).astype(jnp.int32))
        wdim = lax.shift_right_logical(jnp.int32(256), lvl - 2)
        wm1 = wdim - 1
        wm1f = wm1.astype(jnp.float32)

        ysv = y1 * wm1f + lanesf * (bh * wm1f * (1.0 / 6.0))
        xsv = x1 * wm1f + lanesf * (bw * wm1f * (1.0 / 6.0))
        y0i = ysv.astype(jnp.int32)
        x0i = xsv.astype(jnp.int32)
        wyv = ysv - y0i.astype(jnp.float32)
        wxv = xsv - x0i.astype(jnp.float32)
        y0c = jnp.maximum(jnp.minimum(y0i, wm1), 0)
        x0c = jnp.maximum(jnp.minimum(x0i, wm1), 0)
        y0t[...] = y0c
        y1t[...] = jnp.minimum(y0c + 1, wm1)
        x0t[...] = x0c
        x1t[...] = jnp.minimum(x0c + 1, wm1)
        for k in range(POOL_H):
            wysp[k, :] = jnp.full((16,), wyv[k])
            wxsp[k, :] = jnp.full((16,), wxv[k])

        for ytab, idxref in ((y0t, idx_a), (y1t, idx_b)):
            for j, off in enumerate(_CHUNK_OFFS):
                iy = ct_v[pl.ds(j * 48, 16)]
                ix = ct_v[pl.ds(j * 48 + 16, 16)]
                is_tl = ct_v[pl.ds(j * 48 + 32, 16)] > 0
                yv = plsc.load_gather(ytab, [iy])
                xv = jnp.where(is_tl,
                               plsc.load_gather(x0t, [ix]),
                               plsc.load_gather(x1t, [ix]))
                idxv = yv * wdim + xv
                idxref[pl.ds(off, 16)] = idxv

        for level, fmap in ((2, f2), (3, f3), (4, f4), (5, f5)):
            @pl.when(lvl == level)
            def _():
                pltpu.async_copy(fmap.at[idx_a], rows_a, sem_g)
                pltpu.async_copy(fmap.at[idx_b], rows_b, sem_g)

    def consume(i, rows_a, rows_b, wysp, wxsp, sem_g):
        pltpu.make_async_copy(f2.at[idx_a0], rows_a, sem_g).wait()
        pltpu.make_async_copy(f2.at[idx_b0], rows_b, sem_g).wait()

        @pl.when(i >= 1)
        def _():
            pltpu.make_async_copy(out_v.at[pl.ds(0, 48)], out_hbm.at[oidx],
                                  sem_o).wait()
            pltpu.make_async_copy(out_v.at[pl.ds(48, 16)], out_hbm.at[oidx2],
                                  sem_o).wait()

        def iy_body(iy, c1):
            wyv = wysp[iy, :]
            omy = 1.0 - wyv

            @plsc.parallel_loop(0, POOL_W, unroll=2)
            def _(ix):
                wxv = wxsp[ix, :]
                omx = 1.0 - wxv
                w00 = omy * omx
                w01 = omy * wxv
                w10 = wyv * omx
                w11 = wyv * wxv
                p = iy * POOL_W + ix
                q = p + NPX
                for ck in range(C // 16):
                    sl = pl.ds(ck * 16, 16)
                    out_v[p, sl] = (rows_a[p, sl] * w00 + rows_a[q, sl] * w01
                                    + rows_b[p, sl] * w10 + rows_b[q, sl] * w11)
            return c1
        lax.fori_loop(0, POOL_H, iy_body, 0)

        for ck in range(C // 16):
            sl = pl.ds(ck * 16, 16)
            v48 = out_v[48, sl]
            for r in range(49, 64):
                out_v[r, sl] = v48
        b = base + i
        for off in (0, 16, 32):
            oidx[pl.ds(off, 16)] = b + (lanes + off) * NBOX
        oidx2[...] = jnp.full((16,), b + (NPX - 1) * NBOX, jnp.int32)
        pltpu.async_copy(out_v.at[pl.ds(0, 48)], out_hbm.at[oidx], sem_o)
        pltpu.async_copy(out_v.at[pl.ds(48, 16)], out_hbm.at[oidx2], sem_o)

    prefetch(0, idx_a0, idx_b0, rows_a0, rows_b0, wy0, wx0, sem_g0)

    def box_body(i, carry):
        par0 = (i & 1) == 0
        nxt = i + 1

        @pl.when(nxt < cnt)
        def _():
            @pl.when(par0)
            def _():
                prefetch(nxt, idx_a1, idx_b1, rows_a1, rows_b1,
                         wy1, wx1, sem_g1)

            @pl.when(jnp.logical_not(par0))
            def _():
                prefetch(nxt, idx_a0, idx_b0, rows_a0, rows_b0,
                         wy0, wx0, sem_g0)

        @pl.when(par0)
        def _():
            consume(i, rows_a0, rows_b0, wy0, wx0, sem_g0)

        @pl.when(jnp.logical_not(par0))
        def _():
            consume(i, rows_a1, rows_b1, wy1, wx1, sem_g1)

        return carry

    lax.fori_loop(0, cnt, box_body, 0)
    pltpu.make_async_copy(out_v.at[pl.ds(0, 48)], out_hbm.at[oidx],
                          sem_o).wait()
    pltpu.make_async_copy(out_v.at[pl.ds(48, 16)], out_hbm.at[oidx2],
                          sem_o).wait()


def _make_ctab():
    import numpy as np
    rows = []
    for off in _CHUNK_OFFS:
        g = np.minimum(np.arange(off, off + 16), NIDX - 1)
        p = g % NPX
        rows += [p // POOL_W, p % POOL_W, (g < NPX).astype(np.int64)]
    return jnp.asarray(np.concatenate(rows), jnp.int32)


@jax.jit
def _roialign(boxes_flat, meta_flat, ctab, f2, f3, f4, f5):
    mesh = plsc.VectorSubcoreMesh(core_axis_name="c", subcore_axis_name="s",
                                  num_cores=2, num_subcores=16)
    return pl.kernel(
        _roi_body,
        out_type=jax.ShapeDtypeStruct((NPX * NBOX, C), jnp.float32),
        mesh=mesh,
        scratch_types=[
            pltpu.VMEM((BPW * 8 + 16,), jnp.float32),
            pltpu.VMEM((96,), jnp.float32),
            pltpu.VMEM((336,), jnp.int32),
            pltpu.VMEM((16,), jnp.int32),
            pltpu.VMEM((16,), jnp.int32),
            pltpu.VMEM((16,), jnp.int32),
            pltpu.VMEM((16,), jnp.int32),
            pltpu.VMEM((8, 16), jnp.float32),
            pltpu.VMEM((8, 16), jnp.float32),
            pltpu.VMEM((8, 16), jnp.float32),
            pltpu.VMEM((8, 16), jnp.float32),
            pltpu.VMEM((NIDXP,), jnp.int32),
            pltpu.VMEM((NIDXP,), jnp.int32),
            pltpu.VMEM((NIDXP,), jnp.int32),
            pltpu.VMEM((NIDXP,), jnp.int32),
            pltpu.VMEM((48,), jnp.int32),
            pltpu.VMEM((16,), jnp.int32),
            pltpu.VMEM((NIDXP, C), jnp.float32),
            pltpu.VMEM((NIDXP, C), jnp.float32),
            pltpu.VMEM((NIDXP, C), jnp.float32),
            pltpu.VMEM((NIDXP, C), jnp.float32),
            pltpu.VMEM((64, C), jnp.float32),
            pltpu.SemaphoreType.DMA,
            pltpu.SemaphoreType.DMA,
            pltpu.SemaphoreType.DMA,
        ],
        compiler_params=pltpu.CompilerParams(use_tc_tiling_on_sc=True,
                                             needs_layout_passes=False),
    )(boxes_flat, meta_flat, ctab, f2, f3, f4, f5)


def kernel(boxes, image_meta, p2, p3, p4, p5):
    n = boxes.shape[1]
    boxes8 = jnp.pad(boxes.reshape(-1, 4), ((0, 1026 - n), (0, 4)))
    meta_flat = jnp.pad(image_meta.reshape(-1), (0, 96 - image_meta.size))
    out = _roialign(boxes8.reshape(-1), meta_flat, _make_ctab(),
                    p2.reshape(-1, C), p3.reshape(-1, C),
                    p4.reshape(-1, C), p5.reshape(-1, C))
    out5 = out.reshape(1, POOL_H, POOL_W, n, C)
    return jnp.transpose(out5, (0, 3, 1, 2, 4))


# speedup vs baseline: 3.4068x; 1.0041x over previous
import jax
import jax.numpy as jnp
from jax import lax
from jax.experimental import pallas as pl
from jax.experimental.pallas import tpu as pltpu
from jax.experimental.pallas import tpu_sc as plsc

POOL_H = 7
POOL_W = 7
NPX = POOL_H * POOL_W
NIDX = 2 * NPX
C = 256
NW = 32
NBOX = 1000
BPW = 32
NIDXP = 104
_CHUNK_OFFS = (0, 16, 32, 48, 64, 80, 88)


def _roi_body(boxes_hbm, meta_hbm, ctab_hbm, f2, f3, f4, f5, out_hbm,
              bx_v, meta_v, ct_v, y0t, y1t, x0t, x1t,
              wy0, wx0, wy1, wx1,
              idx_a0, idx_b0, idx_a1, idx_b1, oidx, oidx2,
              rows_a0, rows_b0, rows_a1, rows_b1,
              out_v, sem_g0, sem_g1, sem_o):
    cid = lax.axis_index("c")
    sid = lax.axis_index("s")
    wid = sid * 2 + cid
    base = wid * 31 + jnp.minimum(wid, 8)
    cnt = 31 + (wid < 8).astype(jnp.int32)

    pltpu.sync_copy(boxes_hbm.at[pl.ds(base * 8, BPW * 8 + 16)], bx_v)
    pltpu.sync_copy(meta_hbm, meta_v)
    pltpu.sync_copy(ctab_hbm, ct_v)
    mv = meta_v[pl.ds(0, 16)]
    area = mv[4] * mv[5]
    th3 = jnp.float32(224.0 * 224.0 * 0.125)
    th4 = jnp.float32(224.0 * 224.0 * 0.5)
    th5 = jnp.float32(224.0 * 224.0 * 2.0)
    lanes = lax.broadcasted_iota(jnp.int32, (16,), 0)
    lanesf = lanes.astype(jnp.float32)

    def prefetch(i, idx_a, idx_b, rows_a, rows_b, wysp, wxsp, sem_g):
        bv = bx_v[pl.ds(i * 8, 16)]
        y1 = bv[0]
        x1 = bv[1]
        y2 = bv[2]
        x2 = bv[3]
        bh = y2 - y1
        bw = x2 - x1
        hw = bh * bw * area
        lvl = (2 + (hw > th3).astype(jnp.int32)
               + (hw > th4).astype(jnp.int32)
               + (hw > th5).astype(jnp.int32))
        wdim = lax.shift_right_logical(jnp.int32(256), lvl - 2)
        wm1 = wdim - 1
        wm1f = wm1.astype(jnp.float32)

        ysv = y1 * wm1f + lanesf * (bh * wm1f * (1.0 / 6.0))
        xsv = x1 * wm1f + lanesf * (bw * wm1f * (1.0 / 6.0))
        y0i = ysv.astype(jnp.int32)
        x0i = xsv.astype(jnp.int32)
        wyv = ysv - y0i.astype(jnp.float32)
        wxv = xsv - x0i.astype(jnp.float32)
        y0c = jnp.maximum(jnp.minimum(y0i, wm1), 0)
        x0c = jnp.maximum(jnp.minimum(x0i, wm1), 0)
        y0t[...] = y0c
        y1t[...] = jnp.minimum(y0c + 1, wm1)
        x0t[...] = x0c
        x1t[...] = jnp.minimum(x0c + 1, wm1)
        for k in range(POOL_H):
            wysp[k, :] = jnp.full((16,), wyv[k])
            wxsp[k, :] = jnp.full((16,), wxv[k])

        for ytab, idxref in ((y0t, idx_a), (y1t, idx_b)):
            for j, off in enumerate(_CHUNK_OFFS):
                iy = ct_v[pl.ds(j * 48, 16)]
                ix = ct_v[pl.ds(j * 48 + 16, 16)]
                is_tl = ct_v[pl.ds(j * 48 + 32, 16)] > 0
                yv = plsc.load_gather(ytab, [iy])
                xv = jnp.where(is_tl,
                               plsc.load_gather(x0t, [ix]),
                               plsc.load_gather(x1t, [ix]))
                idxv = yv * wdim + xv
                idxref[pl.ds(off, 16)] = idxv

        for level, fmap in ((2, f2), (3, f3), (4, f4), (5, f5)):
            @pl.when(lvl == level)
            def _():
                pltpu.async_copy(fmap.at[idx_a], rows_a, sem_g)
                pltpu.async_copy(fmap.at[idx_b], rows_b, sem_g)

    def consume(i, rows_a, rows_b, wysp, wxsp, sem_g):
        pltpu.make_async_copy(f2.at[idx_a0], rows_a, sem_g).wait()
        pltpu.make_async_copy(f2.at[idx_b0], rows_b, sem_g).wait()

        @pl.when(i >= 1)
        def _():
            pltpu.make_async_copy(out_v.at[pl.ds(0, 48)], out_hbm.at[oidx],
                                  sem_o).wait()
            pltpu.make_async_copy(out_v.at[pl.ds(48, 16)], out_hbm.at[oidx2],
                                  sem_o).wait()

        @plsc.parallel_loop(0, POOL_H)
        def _(iy):
            wyv = wysp[iy, :]
            omy = 1.0 - wyv

            @plsc.parallel_loop(0, POOL_W, unroll=2)
            def _(ix):
                wxv = wxsp[ix, :]
                omx = 1.0 - wxv
                w00 = omy * omx
                w01 = omy * wxv
                w10 = wyv * omx
                w11 = wyv * wxv
                p = iy * POOL_W + ix
                q = p + NPX
                for ck in range(C // 16):
                    sl = pl.ds(ck * 16, 16)
                    out_v[p, sl] = (rows_a[p, sl] * w00 + rows_a[q, sl] * w01
                                    + rows_b[p, sl] * w10 + rows_b[q, sl] * w11)

        for ck in range(C // 16):
            sl = pl.ds(ck * 16, 16)
            v48 = out_v[48, sl]
            for r in range(49, 64):
                out_v[r, sl] = v48
        b = base + i
        for off in (0, 16, 32):
            oidx[pl.ds(off, 16)] = b + (lanes + off) * NBOX
        oidx2[...] = jnp.full((16,), b + (NPX - 1) * NBOX, jnp.int32)
        pltpu.async_copy(out_v.at[pl.ds(0, 48)], out_hbm.at[oidx], sem_o)
        pltpu.async_copy(out_v.at[pl.ds(48, 16)], out_hbm.at[oidx2], sem_o)

    prefetch(0, idx_a0, idx_b0, rows_a0, rows_b0, wy0, wx0, sem_g0)

    def box_body(i, carry):
        par0 = (i & 1) == 0
        nxt = i + 1

        @pl.when(nxt < cnt)
        def _():
            @pl.when(par0)
            def _():
                prefetch(nxt, idx_a1, idx_b1, rows_a1, rows_b1,
                         wy1, wx1, sem_g1)

            @pl.when(jnp.logical_not(par0))
            def _():
                prefetch(nxt, idx_a0, idx_b0, rows_a0, rows_b0,
                         wy0, wx0, sem_g0)

        @pl.when(par0)
        def _():
            consume(i, rows_a0, rows_b0, wy0, wx0, sem_g0)

        @pl.when(jnp.logical_not(par0))
        def _():
            consume(i, rows_a1, rows_b1, wy1, wx1, sem_g1)

        return carry

    lax.fori_loop(0, cnt, box_body, 0)
    pltpu.make_async_copy(out_v.at[pl.ds(0, 48)], out_hbm.at[oidx],
                          sem_o).wait()
    pltpu.make_async_copy(out_v.at[pl.ds(48, 16)], out_hbm.at[oidx2],
                          sem_o).wait()


def _make_ctab():
    import numpy as np
    rows = []
    for off in _CHUNK_OFFS:
        g = np.minimum(np.arange(off, off + 16), NIDX - 1)
        p = g % NPX
        rows += [p // POOL_W, p % POOL_W, (g < NPX).astype(np.int64)]
    return jnp.asarray(np.concatenate(rows), jnp.int32)


@jax.jit
def _roialign(boxes_flat, meta_flat, ctab, f2, f3, f4, f5):
    mesh = plsc.VectorSubcoreMesh(core_axis_name="c", subcore_axis_name="s",
                                  num_cores=2, num_subcores=16)
    return pl.kernel(
        _roi_body,
        out_type=jax.ShapeDtypeStruct((NPX * NBOX, C), jnp.float32),
        mesh=mesh,
        scratch_types=[
            pltpu.VMEM((BPW * 8 + 16,), jnp.float32),
            pltpu.VMEM((96,), jnp.float32),
            pltpu.VMEM((336,), jnp.int32),
            pltpu.VMEM((16,), jnp.int32),
            pltpu.VMEM((16,), jnp.int32),
            pltpu.VMEM((16,), jnp.int32),
            pltpu.VMEM((16,), jnp.int32),
            pltpu.VMEM((8, 16), jnp.float32),
            pltpu.VMEM((8, 16), jnp.float32),
            pltpu.VMEM((8, 16), jnp.float32),
            pltpu.VMEM((8, 16), jnp.float32),
            pltpu.VMEM((NIDXP,), jnp.int32),
            pltpu.VMEM((NIDXP,), jnp.int32),
            pltpu.VMEM((NIDXP,), jnp.int32),
            pltpu.VMEM((NIDXP,), jnp.int32),
            pltpu.VMEM((48,), jnp.int32),
            pltpu.VMEM((16,), jnp.int32),
            pltpu.VMEM((NIDXP, C), jnp.float32),
            pltpu.VMEM((NIDXP, C), jnp.float32),
            pltpu.VMEM((NIDXP, C), jnp.float32),
            pltpu.VMEM((NIDXP, C), jnp.float32),
            pltpu.VMEM((64, C), jnp.float32),
            pltpu.SemaphoreType.DMA,
            pltpu.SemaphoreType.DMA,
            pltpu.SemaphoreType.DMA,
        ],
        compiler_params=pltpu.CompilerParams(use_tc_tiling_on_sc=True,
                                             needs_layout_passes=False),
    )(boxes_flat, meta_flat, ctab, f2, f3, f4, f5)


def kernel(boxes, image_meta, p2, p3, p4, p5):
    n = boxes.shape[1]
    boxes8 = jnp.pad(boxes.reshape(-1, 4), ((0, 1026 - n), (0, 4)))
    meta_flat = jnp.pad(image_meta.reshape(-1), (0, 96 - image_meta.size))
    out = _roialign(boxes8.reshape(-1), meta_flat, _make_ctab(),
                    p2.reshape(-1, C), p3.reshape(-1, C),
                    p4.reshape(-1, C), p5.reshape(-1, C))
    out5 = out.reshape(1, POOL_H, POOL_W, n, C)
    return jnp.transpose(out5, (0, 3, 1, 2, 4))


# speedup vs baseline: 3.4875x; 1.0237x over previous
import jax
import jax.numpy as jnp
from jax import lax
from jax.experimental import pallas as pl
from jax.experimental.pallas import tpu as pltpu
from jax.experimental.pallas import tpu_sc as plsc

POOL_H = 7
POOL_W = 7
NPX = POOL_H * POOL_W
NIDX = 2 * NPX
C = 256
NW = 32
NBOX = 1000
BPW = 32
NIDXP = 104
_CHUNK_OFFS = (0, 16, 32, 48, 64, 80, 88)


def _roi_body(boxes_hbm, meta_hbm, ctab_hbm, f2, f3, f4, f5, out_hbm,
              bx_v, meta_v, ct_v, y0t, y1t, x0t, x1t,
              wy0, wx0, wy1, wx1,
              idx_a0, idx_b0, idx_a1, idx_b1, oidx, oidx2,
              rows_a0, rows_b0, rows_a1, rows_b1,
              out_v, sem_g0, sem_g1, sem_o):
    cid = lax.axis_index("c")
    sid = lax.axis_index("s")
    wid = sid * 2 + cid
    base = wid * 31 + jnp.minimum(wid, 8)
    cnt = 31 + (wid < 8).astype(jnp.int32)

    pltpu.sync_copy(boxes_hbm.at[pl.ds(base * 8, BPW * 8 + 16)], bx_v)
    pltpu.sync_copy(meta_hbm, meta_v)
    pltpu.sync_copy(ctab_hbm, ct_v)
    mv = meta_v[pl.ds(0, 16)]
    area = mv[4] * mv[5]
    th3 = jnp.float32(224.0 * 224.0 * 0.125)
    th4 = jnp.float32(224.0 * 224.0 * 0.5)
    th5 = jnp.float32(224.0 * 224.0 * 2.0)
    lanes = lax.broadcasted_iota(jnp.int32, (16,), 0)
    lanesf = lanes.astype(jnp.float32)

    def prefetch(i, idx_a, idx_b, rows_a, rows_b, wysp, wxsp, sem_g):
        bv = bx_v[pl.ds(i * 8, 16)]
        y1 = bv[0]
        x1 = bv[1]
        y2 = bv[2]
        x2 = bv[3]
        bh = y2 - y1
        bw = x2 - x1
        hw = bh * bw * area
        lvl = (2 + (hw > th3).astype(jnp.int32)
               + (hw > th4).astype(jnp.int32)
               + (hw > th5).astype(jnp.int32))
        wdim = lax.shift_right_logical(jnp.int32(256), lvl - 2)
        wm1 = wdim - 1
        wm1f = wm1.astype(jnp.float32)

        ysv = y1 * wm1f + lanesf * (bh * wm1f * (1.0 / 6.0))
        xsv = x1 * wm1f + lanesf * (bw * wm1f * (1.0 / 6.0))
        y0i = ysv.astype(jnp.int32)
        x0i = xsv.astype(jnp.int32)
        wyv = ysv - y0i.astype(jnp.float32)
        wxv = xsv - x0i.astype(jnp.float32)
        y0c = jnp.maximum(jnp.minimum(y0i, wm1), 0)
        x0c = jnp.maximum(jnp.minimum(x0i, wm1), 0)
        y0t[...] = y0c
        y1t[...] = jnp.minimum(y0c + 1, wm1)
        x0t[...] = x0c
        x1t[...] = jnp.minimum(x0c + 1, wm1)
        for k in range(POOL_H):
            wysp[k, :] = jnp.full((16,), wyv[k])
            wxsp[k, :] = jnp.full((16,), wxv[k])

        for ytab, idxref in ((y0t, idx_a), (y1t, idx_b)):
            for j, off in enumerate(_CHUNK_OFFS):
                iy = ct_v[pl.ds(j * 48, 16)]
                ix = ct_v[pl.ds(j * 48 + 16, 16)]
                is_tl = ct_v[pl.ds(j * 48 + 32, 16)] > 0
                yv = plsc.load_gather(ytab, [iy])
                xv = jnp.where(is_tl,
                               plsc.load_gather(x0t, [ix]),
                               plsc.load_gather(x1t, [ix]))
                idxv = yv * wdim + xv
                idxref[pl.ds(off, 16)] = idxv

        for level, fmap in ((2, f2), (3, f3), (4, f4), (5, f5)):
            @pl.when(lvl == level)
            def _():
                pltpu.async_copy(fmap.at[idx_a], rows_a, sem_g)
                pltpu.async_copy(fmap.at[idx_b], rows_b, sem_g)

    def consume(i, rows_a, rows_b, wysp, wxsp, sem_g):
        pltpu.make_async_copy(f2.at[idx_a0], rows_a, sem_g).wait()
        pltpu.make_async_copy(f2.at[idx_b0], rows_b, sem_g).wait()

        @pl.when(i >= 1)
        def _():
            pltpu.make_async_copy(out_v.at[pl.ds(0, 48)], out_hbm.at[oidx],
                                  sem_o).wait()
            pltpu.make_async_copy(out_v.at[pl.ds(48, 16)], out_hbm.at[oidx2],
                                  sem_o).wait()

        @plsc.parallel_loop(0, POOL_H)
        def _(iy):
            wyv = wysp[iy, :]
            omy = 1.0 - wyv

            @plsc.parallel_loop(0, POOL_W, unroll=3)
            def _(ix):
                wxv = wxsp[ix, :]
                omx = 1.0 - wxv
                w00 = omy * omx
                w01 = omy * wxv
                w10 = wyv * omx
                w11 = wyv * wxv
                p = iy * POOL_W + ix
                q = p + NPX
                for ck in range(C // 16):
                    sl = pl.ds(ck * 16, 16)
                    out_v[p, sl] = (rows_a[p, sl] * w00 + rows_a[q, sl] * w01
                                    + rows_b[p, sl] * w10 + rows_b[q, sl] * w11)

        for ck in range(C // 16):
            sl = pl.ds(ck * 16, 16)
            v48 = out_v[48, sl]
            for r in range(49, 64):
                out_v[r, sl] = v48
        b = base + i
        for off in (0, 16, 32):
            oidx[pl.ds(off, 16)] = b + (lanes + off) * NBOX
        oidx2[...] = jnp.full((16,), b + (NPX - 1) * NBOX, jnp.int32)
        pltpu.async_copy(out_v.at[pl.ds(0, 48)], out_hbm.at[oidx], sem_o)
        pltpu.async_copy(out_v.at[pl.ds(48, 16)], out_hbm.at[oidx2], sem_o)

    prefetch(0, idx_a0, idx_b0, rows_a0, rows_b0, wy0, wx0, sem_g0)

    def box_body(i, carry):
        par0 = (i & 1) == 0
        nxt = i + 1

        @pl.when(nxt < cnt)
        def _():
            @pl.when(par0)
            def _():
                prefetch(nxt, idx_a1, idx_b1, rows_a1, rows_b1,
                         wy1, wx1, sem_g1)

            @pl.when(jnp.logical_not(par0))
            def _():
                prefetch(nxt, idx_a0, idx_b0, rows_a0, rows_b0,
                         wy0, wx0, sem_g0)

        @pl.when(par0)
        def _():
            consume(i, rows_a0, rows_b0, wy0, wx0, sem_g0)

        @pl.when(jnp.logical_not(par0))
        def _():
            consume(i, rows_a1, rows_b1, wy1, wx1, sem_g1)

        return carry

    lax.fori_loop(0, cnt, box_body, 0)
    pltpu.make_async_copy(out_v.at[pl.ds(0, 48)], out_hbm.at[oidx],
                          sem_o).wait()
    pltpu.make_async_copy(out_v.at[pl.ds(48, 16)], out_hbm.at[oidx2],
                          sem_o).wait()


def _make_ctab():
    import numpy as np
    rows = []
    for off in _CHUNK_OFFS:
        g = np.minimum(np.arange(off, off + 16), NIDX - 1)
        p = g % NPX
        rows += [p // POOL_W, p % POOL_W, (g < NPX).astype(np.int64)]
    return jnp.asarray(np.concatenate(rows), jnp.int32)


@jax.jit
def _roialign(boxes_flat, meta_flat, ctab, f2, f3, f4, f5):
    mesh = plsc.VectorSubcoreMesh(core_axis_name="c", subcore_axis_name="s",
                                  num_cores=2, num_subcores=16)
    return pl.kernel(
        _roi_body,
        out_type=jax.ShapeDtypeStruct((NPX * NBOX, C), jnp.float32),
        mesh=mesh,
        scratch_types=[
            pltpu.VMEM((BPW * 8 + 16,), jnp.float32),
            pltpu.VMEM((96,), jnp.float32),
            pltpu.VMEM((336,), jnp.int32),
            pltpu.VMEM((16,), jnp.int32),
            pltpu.VMEM((16,), jnp.int32),
            pltpu.VMEM((16,), jnp.int32),
            pltpu.VMEM((16,), jnp.int32),
            pltpu.VMEM((8, 16), jnp.float32),
            pltpu.VMEM((8, 16), jnp.float32),
            pltpu.VMEM((8, 16), jnp.float32),
            pltpu.VMEM((8, 16), jnp.float32),
            pltpu.VMEM((NIDXP,), jnp.int32),
            pltpu.VMEM((NIDXP,), jnp.int32),
            pltpu.VMEM((NIDXP,), jnp.int32),
            pltpu.VMEM((NIDXP,), jnp.int32),
            pltpu.VMEM((48,), jnp.int32),
            pltpu.VMEM((16,), jnp.int32),
            pltpu.VMEM((NIDXP, C), jnp.float32),
            pltpu.VMEM((NIDXP, C), jnp.float32),
            pltpu.VMEM((NIDXP, C), jnp.float32),
            pltpu.VMEM((NIDXP, C), jnp.float32),
            pltpu.VMEM((64, C), jnp.float32),
            pltpu.SemaphoreType.DMA,
            pltpu.SemaphoreType.DMA,
            pltpu.SemaphoreType.DMA,
        ],
        compiler_params=pltpu.CompilerParams(use_tc_tiling_on_sc=True,
                                             needs_layout_passes=False),
    )(boxes_flat, meta_flat, ctab, f2, f3, f4, f5)


def kernel(boxes, image_meta, p2, p3, p4, p5):
    n = boxes.shape[1]
    boxes8 = jnp.pad(boxes.reshape(-1, 4), ((0, 1026 - n), (0, 4)))
    meta_flat = jnp.pad(image_meta.reshape(-1), (0, 96 - image_meta.size))
    out = _roialign(boxes8.reshape(-1), meta_flat, _make_ctab(),
                    p2.reshape(-1, C), p3.reshape(-1, C),
                    p4.reshape(-1, C), p5.reshape(-1, C))
    out5 = out.reshape(1, POOL_H, POOL_W, n, C)
    return jnp.transpose(out5, (0, 3, 1, 2, 4))
